# baseline (speedup 1.0000x reference)
import dataclasses
import functools

import jax
import jax.numpy as jnp
from jax import lax
from jax.experimental import pallas as pl
from jax.experimental.pallas import tpu as pltpu
from jax.experimental.pallas import tpu_sc as plsc

N = 10000
E = 160000
D = 256
QW = 64
NQ = D // QW
L = 16
NS = 16
EPT = E // NS
CHUNK = 80
NCHUNKS = EPT // CHUNK
NBUF = 5
RB = 80
NRB = N // RB
RBROUNDS = -(-NRB // NS)
TCB = 1000


def _write_quarters(h, h_ref, als_ref, ald_ref, asrc_ref, adst_ref):
    for q in range(NQ):
        h_ref[q] = h[:, q * QW:(q + 1) * QW]
    als_ref[...] = jnp.dot(h, asrc_ref[...], preferred_element_type=jnp.float32)
    ald_ref[...] = jnp.dot(h, adst_ref[...], preferred_element_type=jnp.float32)


def _tc1_body(x_ref, w_ref, asrc_ref, adst_ref, h_ref, als_ref, ald_ref):
    h = jnp.dot(x_ref[...], w_ref[...], preferred_element_type=jnp.float32)
    _write_quarters(h, h_ref, als_ref, ald_ref, asrc_ref, adst_ref)


def _tc2_body(x0_ref, x1_ref, x2_ref, x3_ref, w_ref, asrc_ref, adst_ref,
              h_ref, als_ref, ald_ref):
    w = w_ref[...]
    h = jnp.dot(x0_ref[...], w[0 * QW:1 * QW], preferred_element_type=jnp.float32)
    h += jnp.dot(x1_ref[...], w[1 * QW:2 * QW], preferred_element_type=jnp.float32)
    h += jnp.dot(x2_ref[...], w[2 * QW:3 * QW], preferred_element_type=jnp.float32)
    h += jnp.dot(x3_ref[...], w[3 * QW:4 * QW], preferred_element_type=jnp.float32)
    _write_quarters(h, h_ref, als_ref, ald_ref, asrc_ref, adst_ref)


_TC_OUT_SPECS = [
    pl.BlockSpec((NQ, TCB, QW), lambda i: (0, i, 0)),
    pl.BlockSpec((TCB, 1), lambda i: (i, 0)),
    pl.BlockSpec((TCB, 1), lambda i: (i, 0)),
]
_TC_OUT_SHAPE = [
    jax.ShapeDtypeStruct((NQ, N, QW), jnp.float32),
    jax.ShapeDtypeStruct((N, 1), jnp.float32),
    jax.ShapeDtypeStruct((N, 1), jnp.float32),
]


def _tc_layer1(x, W, a_src, a_dst):
    return pl.pallas_call(
        _tc1_body,
        grid=(N // TCB,),
        in_specs=[
            pl.BlockSpec((TCB, D), lambda i: (i, 0)),
            pl.BlockSpec((D, D), lambda i: (0, 0)),
            pl.BlockSpec((D, 1), lambda i: (0, 0)),
            pl.BlockSpec((D, 1), lambda i: (0, 0)),
        ],
        out_specs=_TC_OUT_SPECS,
        out_shape=_TC_OUT_SHAPE,
    )(x, W, a_src.reshape(D, 1), a_dst.reshape(D, 1))


def _tc_layer2(o1_flat, W, a_src, a_dst):
    nblk = N // TCB
    in_specs = [
        pl.BlockSpec((TCB, QW), lambda i, q=q, n=nblk: (i + q * n, 0))
        for q in range(NQ)
    ]
    return pl.pallas_call(
        _tc2_body,
        grid=(nblk,),
        in_specs=in_specs + [
            pl.BlockSpec((D, D), lambda i: (0, 0)),
            pl.BlockSpec((D, 1), lambda i: (0, 0)),
            pl.BlockSpec((D, 1), lambda i: (0, 0)),
        ],
        out_specs=_TC_OUT_SPECS,
        out_shape=_TC_OUT_SHAPE,
    )(o1_flat, o1_flat, o1_flat, o1_flat, W,
      a_src.reshape(D, 1), a_dst.reshape(D, 1))


def _sc_edge_kernel(h4, als, ald, src, dst, b):
    mesh = plsc.VectorSubcoreMesh(core_axis_name="c", subcore_axis_name="s")
    cp = pltpu.CompilerParams()
    for field, val in (("needs_layout_passes", False),
                       ("use_tc_tiling_on_sc", False)):
        if field in pltpu.CompilerParams.__dataclass_fields__:
            cp = dataclasses.replace(cp, **{field: val})

    @functools.partial(
        pl.kernel,
        mesh=mesh,
        compiler_params=cp,
        out_type=jax.ShapeDtypeStruct((NQ * N, QW), jnp.float32),
        scratch_types=[
            pltpu.VMEM((N,), jnp.float32),
            pltpu.VMEM((N,), jnp.float32),
            pltpu.VMEM((QW,), jnp.float32),
            pltpu.VMEM((NBUF, CHUNK, QW), jnp.float32),
            pltpu.VMEM((NBUF, CHUNK, L), jnp.float32),
            pltpu.VMEM((EPT,), jnp.int32),
            pltpu.VMEM((NCHUNKS, CHUNK), jnp.int32),
            pltpu.VMEM((RB, QW), jnp.float32),
            pltpu.VMEM((RB, L), jnp.float32),
            pltpu.VMEM_SHARED((N, QW), jnp.float32),
            pltpu.VMEM_SHARED((N, L), jnp.float32),
            pltpu.SemaphoreType.DMA((NBUF,)),
            pltpu.SemaphoreType.DMA((NBUF,)),
            pltpu.SemaphoreType.DMA((NBUF,)),
        ],
    )
    def sck(h_hbm, als_hbm, ald_hbm, src_hbm, dst3_hbm, b_hbm, o_hbm,
            als_v, ald_v, b_v, rows_v, exr_v, src_t, dst2_t,
            stg_v, dstg_v, acc_sh, dacc_sh, sem_g, sem_s, sem_d):
        c = lax.axis_index("c")
        s = lax.axis_index("s")

        pltpu.sync_copy(als_hbm, als_v)
        pltpu.sync_copy(ald_hbm, ald_v)
        pltpu.sync_copy(src_hbm.at[pl.ds(s * EPT, EPT)], src_t)
        pltpu.sync_copy(dst3_hbm.at[s], dst2_t)

        zero = jnp.zeros((L,), jnp.float32)

        @pl.loop(0, RB)
        def _(i):
            for j in range(QW // L):
                stg_v[i, pl.ds(j * L, L)] = zero
            dstg_v[i, pl.ds(0, L)] = zero

        for b in range(NBUF):
            exr_b = exr_v.at[b]

            @pl.loop(0, CHUNK)
            def _(i, exr_b=exr_b):
                exr_b[i, pl.ds(0, L)] = zero

        @pl.loop(0, RBROUNDS)
        def _(z):
            cid = s + z * NS

            @pl.when(cid < NRB)
            def _():
                pltpu.sync_copy(stg_v, acc_sh.at[pl.ds(cid * RB, RB)])
                pltpu.sync_copy(dstg_v, dacc_sh.at[pl.ds(cid * RB, RB)])

        neg = jnp.full((L,), -1e30, jnp.float32)

        def _maxtab(tab):
            def body(i, cur):
                return jnp.maximum(cur, tab[pl.ds(i * L, L)])
            return jnp.max(lax.fori_loop(0, N // L, body, neg))

        m_z = _maxtab(als_v) + _maxtab(ald_v)
        m_bound = jnp.maximum(m_z, 0.2 * m_z)

        plsc.subcore_barrier()

        lane = lax.iota(jnp.int32, L)
        zlane = jnp.zeros((L,), jnp.int32)

        def edge_pass(q, first):
            row_off = (2 * c + q) * N
            add_off = row_off if first else N

            @pl.loop(0, EPT // L)
            def _(i):
                src_t[pl.ds(i * L, L)] = src_t[pl.ds(i * L, L)] + add_off

            def issue_gather(t, b):
                pltpu.async_copy(
                    h_hbm.at[src_t.at[pl.ds(t * CHUNK, CHUNK)]],
                    rows_v.at[b], sem_g.at[b])

            def wait_gather(t, b):
                pltpu.make_async_copy(
                    h_hbm.at[src_t.at[pl.ds(t * CHUNK, CHUNK)]],
                    rows_v.at[b], sem_g.at[b]).wait()

            def wait_scatters(t, b):
                pltpu.make_async_copy(
                    rows_v.at[b], acc_sh.at[dst2_t.at[t]], sem_s.at[b]).wait()
                if first:
                    pltpu.make_async_copy(
                        exr_v.at[b], dacc_sh.at[dst2_t.at[t]],
                        sem_d.at[b]).wait()

            issue_gather(0, 0)
            issue_gather(1, 1)
            issue_gather(2, 2)
            issue_gather(3, 3)

            @pl.loop(0, NCHUNKS // NBUF)
            def _(u):
                for b in range(NBUF):
                    t = u * NBUF + b
                    rows_b = rows_v.at[b]
                    exr_b = exr_v.at[b]
                    for g in range(CHUNK // L):
                        sv = src_t[pl.ds(t * CHUNK + g * L, L)] - row_off
                        dv = dst2_t[t, pl.ds(g * L, L)]
                        z = (plsc.load_gather(als_v, [sv])
                             + plsc.load_gather(ald_v, [dv]))
                        e = jnp.maximum(z, 0.2 * z)
                        ex = jnp.exp(e - m_bound)
                        plsc.store_scatter(exr_b, [lane + g * L, zlane], ex)
                    wait_gather(t, b)

                    @plsc.parallel_loop(0, CHUNK, 1, unroll=4)
                    def _(k, rows_b=rows_b, exr_b=exr_b):
                        exk = exr_b[k, pl.ds(0, L)][0]
                        for j in range(QW // L):
                            rows_b[k, pl.ds(j * L, L)] = (
                                rows_b[k, pl.ds(j * L, L)] * exk)

                    pltpu.sync_copy(rows_b, acc_sh.at[dst2_t.at[t]], add=True)
                    if first:
                        pltpu.sync_copy(exr_b, dacc_sh.at[dst2_t.at[t]],
                                        add=True)

                    v = (b + 4) % NBUF

                    @pl.when(t + 4 < NCHUNKS)
                    def _(t=t, v=v):
                        issue_gather(t + 4, v)

        def readback(q, rezero):
            pltpu.sync_copy(b_hbm.at[pl.ds((2 * c + q) * QW, QW)], b_v)

            @pl.loop(0, RBROUNDS)
            def _(z):
                cid = s + z * NS

                @pl.when(cid < NRB)
                def _():
                    r0 = cid * RB
                    pltpu.sync_copy(acc_sh.at[pl.ds(r0, RB)], stg_v)
                    pltpu.sync_copy(dacc_sh.at[pl.ds(r0, RB)], dstg_v)

                    @pl.loop(0, RB)
                    def _(i):
                        inv = (1.0 / (dstg_v[i, pl.ds(0, L)] + 1e-16))[0]
                        for j in range(QW // L):
                            stg_v[i, pl.ds(j * L, L)] = (
                                stg_v[i, pl.ds(j * L, L)] * inv
                                + b_v[pl.ds(j * L, L)])

                    pltpu.sync_copy(
                        stg_v, o_hbm.at[pl.ds((2 * c + q) * N + r0, RB)])
                    if rezero:
                        zero16 = jnp.zeros((L,), jnp.float32)

                        @pl.loop(0, RB)
                        def _(i):
                            for j in range(QW // L):
                                stg_v[i, pl.ds(j * L, L)] = zero16

                        pltpu.sync_copy(stg_v, acc_sh.at[pl.ds(r0, RB)])

        edge_pass(0, True)
        plsc.subcore_barrier()
        readback(0, rezero=True)
        plsc.subcore_barrier()
        edge_pass(1, False)
        plsc.subcore_barrier()
        readback(1, rezero=False)

    return sck(h4, als, ald, src, dst.reshape(NS, NCHUNKS, CHUNK), b)


def kernel(x, edge_index, W1, a_src1, a_dst1, b1, W2, a_src2, a_dst2, b2):
    src = edge_index[0]
    dst = edge_index[1]

    h1, als1, ald1 = _tc_layer1(x, W1, a_src1, a_dst1)
    o1 = _sc_edge_kernel(h1.reshape(NQ * N, QW), als1.reshape(N),
                         ald1.reshape(N), src, dst, b1)

    h2, als2, ald2 = _tc_layer2(o1, W2, a_src2, a_dst2)
    o2 = _sc_edge_kernel(h2.reshape(NQ * N, QW), als2.reshape(N),
                         ald2.reshape(N), src, dst, b2)

    return jnp.concatenate([o2[q * N:(q + 1) * N] for q in range(NQ)], axis=1)


# speedup vs baseline: 1.0019x; 1.0019x over previous
import dataclasses
import functools

import jax
import jax.numpy as jnp
from jax import lax
from jax.experimental import pallas as pl
from jax.experimental.pallas import tpu as pltpu
from jax.experimental.pallas import tpu_sc as plsc

N = 10000
E = 160000
D = 256
QW = 64
NQ = D // QW
L = 16
NS = 16
EPT = E // NS
CHUNK = 80
NCHUNKS = EPT // CHUNK
NBUF = 5
RB = 80
NRB = N // RB
RBROUNDS = -(-NRB // NS)
TCB = 1000


def _write_quarters(h, h_ref, als_ref, ald_ref, asrc_ref, adst_ref):
    for q in range(NQ):
        h_ref[q] = h[:, q * QW:(q + 1) * QW]
    als_ref[...] = jnp.dot(h, asrc_ref[...], preferred_element_type=jnp.float32)
    ald_ref[...] = jnp.dot(h, adst_ref[...], preferred_element_type=jnp.float32)


def _tc1_body(x_ref, w_ref, asrc_ref, adst_ref, h_ref, als_ref, ald_ref):
    h = jnp.dot(x_ref[...], w_ref[...], preferred_element_type=jnp.float32)
    _write_quarters(h, h_ref, als_ref, ald_ref, asrc_ref, adst_ref)


def _tc2_body(x0_ref, x1_ref, x2_ref, x3_ref, w_ref, asrc_ref, adst_ref,
              h_ref, als_ref, ald_ref):
    w = w_ref[...]
    h = jnp.dot(x0_ref[...], w[0 * QW:1 * QW], preferred_element_type=jnp.float32)
    h += jnp.dot(x1_ref[...], w[1 * QW:2 * QW], preferred_element_type=jnp.float32)
    h += jnp.dot(x2_ref[...], w[2 * QW:3 * QW], preferred_element_type=jnp.float32)
    h += jnp.dot(x3_ref[...], w[3 * QW:4 * QW], preferred_element_type=jnp.float32)
    _write_quarters(h, h_ref, als_ref, ald_ref, asrc_ref, adst_ref)


_TC_OUT_SPECS = [
    pl.BlockSpec((NQ, TCB, QW), lambda i: (0, i, 0)),
    pl.BlockSpec((TCB, 1), lambda i: (i, 0)),
    pl.BlockSpec((TCB, 1), lambda i: (i, 0)),
]
_TC_OUT_SHAPE = [
    jax.ShapeDtypeStruct((NQ, N, QW), jnp.float32),
    jax.ShapeDtypeStruct((N, 1), jnp.float32),
    jax.ShapeDtypeStruct((N, 1), jnp.float32),
]


def _tc_layer1(x, W, a_src, a_dst):
    return pl.pallas_call(
        _tc1_body,
        grid=(N // TCB,),
        in_specs=[
            pl.BlockSpec((TCB, D), lambda i: (i, 0)),
            pl.BlockSpec((D, D), lambda i: (0, 0)),
            pl.BlockSpec((D, 1), lambda i: (0, 0)),
            pl.BlockSpec((D, 1), lambda i: (0, 0)),
        ],
        out_specs=_TC_OUT_SPECS,
        out_shape=_TC_OUT_SHAPE,
    )(x, W, a_src.reshape(D, 1), a_dst.reshape(D, 1))


def _tc_layer2(o1_flat, W, a_src, a_dst):
    nblk = N // TCB
    in_specs = [
        pl.BlockSpec((TCB, QW), lambda i, q=q, n=nblk: (i + q * n, 0))
        for q in range(NQ)
    ]
    return pl.pallas_call(
        _tc2_body,
        grid=(nblk,),
        in_specs=in_specs + [
            pl.BlockSpec((D, D), lambda i: (0, 0)),
            pl.BlockSpec((D, 1), lambda i: (0, 0)),
            pl.BlockSpec((D, 1), lambda i: (0, 0)),
        ],
        out_specs=_TC_OUT_SPECS,
        out_shape=_TC_OUT_SHAPE,
    )(o1_flat, o1_flat, o1_flat, o1_flat, W,
      a_src.reshape(D, 1), a_dst.reshape(D, 1))


def _sc_edge_kernel(h4, als, ald, src, dst, b):
    mesh = plsc.VectorSubcoreMesh(core_axis_name="c", subcore_axis_name="s")
    cp = pltpu.CompilerParams()
    for field, val in (("needs_layout_passes", False),
                       ("use_tc_tiling_on_sc", False)):
        if field in pltpu.CompilerParams.__dataclass_fields__:
            cp = dataclasses.replace(cp, **{field: val})

    @functools.partial(
        pl.kernel,
        mesh=mesh,
        compiler_params=cp,
        out_type=jax.ShapeDtypeStruct((NQ * N, QW), jnp.float32),
        scratch_types=[
            pltpu.VMEM((N,), jnp.float32),
            pltpu.VMEM((N,), jnp.float32),
            pltpu.VMEM((QW,), jnp.float32),
            pltpu.VMEM((NBUF, CHUNK, QW), jnp.float32),
            pltpu.VMEM((NBUF, CHUNK, L), jnp.float32),
            pltpu.VMEM((EPT,), jnp.int32),
            pltpu.VMEM((NCHUNKS, CHUNK), jnp.int32),
            pltpu.VMEM((RB, QW), jnp.float32),
            pltpu.VMEM((RB, L), jnp.float32),
            pltpu.VMEM_SHARED((N, QW), jnp.float32),
            pltpu.VMEM_SHARED((N, L), jnp.float32),
            pltpu.SemaphoreType.DMA((NBUF,)),
        ],
    )
    def sck(h_hbm, als_hbm, ald_hbm, src_hbm, dst3_hbm, b_hbm, o_hbm,
            als_v, ald_v, b_v, rows_v, exr_v, src_t, dst2_t,
            stg_v, dstg_v, acc_sh, dacc_sh, sem_g):
        c = lax.axis_index("c")
        s = lax.axis_index("s")

        pltpu.sync_copy(als_hbm, als_v)
        pltpu.sync_copy(ald_hbm, ald_v)
        pltpu.sync_copy(src_hbm.at[pl.ds(s * EPT, EPT)], src_t)
        pltpu.sync_copy(dst3_hbm.at[s], dst2_t)

        zero = jnp.zeros((L,), jnp.float32)

        @pl.loop(0, RB)
        def _(i):
            for j in range(QW // L):
                stg_v[i, pl.ds(j * L, L)] = zero
            dstg_v[i, pl.ds(0, L)] = zero

        for b in range(NBUF):
            exr_b = exr_v.at[b]

            @pl.loop(0, CHUNK)
            def _(i, exr_b=exr_b):
                exr_b[i, pl.ds(0, L)] = zero

        @pl.loop(0, RBROUNDS)
        def _(z):
            cid = s + z * NS

            @pl.when(cid < NRB)
            def _():
                pltpu.sync_copy(stg_v, acc_sh.at[pl.ds(cid * RB, RB)])
                pltpu.sync_copy(dstg_v, dacc_sh.at[pl.ds(cid * RB, RB)])

        neg = jnp.full((L,), -1e30, jnp.float32)

        def _maxtab(tab):
            def body(i, cur):
                return jnp.maximum(cur, tab[pl.ds(i * L, L)])
            return jnp.max(lax.fori_loop(0, N // L, body, neg))

        m_z = _maxtab(als_v) + _maxtab(ald_v)
        m_bound = jnp.maximum(m_z, 0.2 * m_z)

        plsc.subcore_barrier()

        lane = lax.iota(jnp.int32, L)
        zlane = jnp.zeros((L,), jnp.int32)

        def edge_pass(q, first):
            row_off = (2 * c + q) * N
            add_off = row_off if first else N

            @pl.loop(0, EPT // L)
            def _(i):
                src_t[pl.ds(i * L, L)] = src_t[pl.ds(i * L, L)] + add_off

            def issue_gather(t, b):
                pltpu.async_copy(
                    h_hbm.at[src_t.at[pl.ds(t * CHUNK, CHUNK)]],
                    rows_v.at[b], sem_g.at[b])

            def wait_gather(t, b):
                pltpu.make_async_copy(
                    h_hbm.at[src_t.at[pl.ds(t * CHUNK, CHUNK)]],
                    rows_v.at[b], sem_g.at[b]).wait()

            def wait_scatters(t, b):
                pltpu.make_async_copy(
                    rows_v.at[b], acc_sh.at[dst2_t.at[t]], sem_s.at[b]).wait()
                if first:
                    pltpu.make_async_copy(
                        exr_v.at[b], dacc_sh.at[dst2_t.at[t]],
                        sem_d.at[b]).wait()

            issue_gather(0, 0)
            issue_gather(1, 1)
            issue_gather(2, 2)

            @pl.loop(0, NCHUNKS // NBUF)
            def _(u):
                for b in range(NBUF):
                    t = u * NBUF + b
                    rows_b = rows_v.at[b]
                    exr_b = exr_v.at[b]
                    for g in range(CHUNK // L):
                        sv = src_t[pl.ds(t * CHUNK + g * L, L)] - row_off
                        dv = dst2_t[t, pl.ds(g * L, L)]
                        z = (plsc.load_gather(als_v, [sv])
                             + plsc.load_gather(ald_v, [dv]))
                        e = jnp.maximum(z, 0.2 * z)
                        ex = jnp.exp(e - m_bound)
                        plsc.store_scatter(exr_b, [lane + g * L, zlane], ex)
                    wait_gather(t, b)

                    @plsc.parallel_loop(0, CHUNK, 1, unroll=4)
                    def _(k, rows_b=rows_b, exr_b=exr_b):
                        exk = exr_b[k, pl.ds(0, L)][0]
                        for j in range(QW // L):
                            rows_b[k, pl.ds(j * L, L)] = (
                                rows_b[k, pl.ds(j * L, L)] * exk)

                    pltpu.sync_copy(rows_b, acc_sh.at[dst2_t.at[t]], add=True)
                    if first:
                        pltpu.sync_copy(exr_b, dacc_sh.at[dst2_t.at[t]],
                                        add=True)

                    v = (b + 3) % NBUF

                    @pl.when(t + 3 < NCHUNKS)
                    def _(t=t, v=v):
                        issue_gather(t + 3, v)

        def readback(q, rezero):
            pltpu.sync_copy(b_hbm.at[pl.ds((2 * c + q) * QW, QW)], b_v)

            @pl.loop(0, RBROUNDS)
            def _(z):
                cid = s + z * NS

                @pl.when(cid < NRB)
                def _():
                    r0 = cid * RB
                    pltpu.sync_copy(acc_sh.at[pl.ds(r0, RB)], stg_v)
                    pltpu.sync_copy(dacc_sh.at[pl.ds(r0, RB)], dstg_v)

                    @pl.loop(0, RB)
                    def _(i):
                        inv = (1.0 / (dstg_v[i, pl.ds(0, L)] + 1e-16))[0]
                        for j in range(QW // L):
                            stg_v[i, pl.ds(j * L, L)] = (
                                stg_v[i, pl.ds(j * L, L)] * inv
                                + b_v[pl.ds(j * L, L)])

                    pltpu.sync_copy(
                        stg_v, o_hbm.at[pl.ds((2 * c + q) * N + r0, RB)])
                    if rezero:
                        zero16 = jnp.zeros((L,), jnp.float32)

                        @pl.loop(0, RB)
                        def _(i):
                            for j in range(QW // L):
                                stg_v[i, pl.ds(j * L, L)] = zero16

                        pltpu.sync_copy(stg_v, acc_sh.at[pl.ds(r0, RB)])

        edge_pass(0, True)
        plsc.subcore_barrier()
        readback(0, rezero=True)
        plsc.subcore_barrier()
        edge_pass(1, False)
        plsc.subcore_barrier()
        readback(1, rezero=False)

    return sck(h4, als, ald, src, dst.reshape(NS, NCHUNKS, CHUNK), b)


def kernel(x, edge_index, W1, a_src1, a_dst1, b1, W2, a_src2, a_dst2, b2):
    src = edge_index[0]
    dst = edge_index[1]

    h1, als1, ald1 = _tc_layer1(x, W1, a_src1, a_dst1)
    o1 = _sc_edge_kernel(h1.reshape(NQ * N, QW), als1.reshape(N),
                         ald1.reshape(N), src, dst, b1)

    h2, als2, ald2 = _tc_layer2(o1, W2, a_src2, a_dst2)
    o2 = _sc_edge_kernel(h2.reshape(NQ * N, QW), als2.reshape(N),
                         ald2.reshape(N), src, dst, b2)

    return jnp.concatenate([o2[q * N:(q + 1) * N] for q in range(NQ)], axis=1)
